# Initial kernel scaffold; baseline (speedup 1.0000x reference)
import jax
import jax.numpy as jnp
from jax.experimental import pallas as pl


def kernel(clsfea, denfea, anchor_cls, bs_mean, b, W_emb, b_emb, W_emb2, b_emb2):
    raise NotImplementedError("write your pallas kernel here")


# speedup vs baseline: 724.7281x; 724.7281x over previous
import functools

import jax
import jax.numpy as jnp
from jax.experimental import pallas as pl
from jax.experimental.pallas import tpu as pltpu


def _select_body(size_ref, cls_ref, den_ref, anc_ref, out_ref):
    cls = cls_ref[0]
    den = den_ref[0]
    anc = anc_ref[0]
    size_f = size_ref[0]

    eps = jnp.float32(1e-12)
    ssq = jnp.sum(cls * cls, axis=0, keepdims=True)
    inv_n = 1.0 / jnp.maximum(jnp.sqrt(ssq), eps)
    asq = jnp.sum(anc * anc, axis=1, keepdims=True)
    na = anc / jnp.maximum(jnp.sqrt(asq), eps)
    na_col = na.reshape(anc.shape[1], 1)

    sim = den * ((cls * inv_n) * na_col)

    bits = pltpu.bitcast(sim, jnp.uint32)
    ku = jnp.where(
        (bits >> 31) == jnp.uint32(0),
        bits | jnp.uint32(0x80000000),
        ~bits,
    )

    C = cls.shape[0]
    T = jnp.zeros((C, 1), jnp.uint32)
    for i in range(32):
        cand = T | jnp.uint32(1 << (31 - i))
        cnt = jnp.sum(jnp.where(ku >= cand, 1.0, 0.0), axis=1, keepdims=True)
        T = jnp.where(cnt >= size_f, cand, T)

    m_ge = ku >= T
    m_gt = ku > T
    sum_ge = jnp.sum(jnp.where(m_ge, cls, 0.0), axis=1, keepdims=True)
    sum_gt = jnp.sum(jnp.where(m_gt, cls, 0.0), axis=1, keepdims=True)
    cnt_ge = jnp.sum(jnp.where(m_ge, 1.0, 0.0), axis=1, keepdims=True)
    cnt_gt = jnp.sum(jnp.where(m_gt, 1.0, 0.0), axis=1, keepdims=True)
    eq_cnt = jnp.maximum(cnt_ge - cnt_gt, 1.0)
    need = size_f - cnt_gt
    row_sum = sum_gt + (sum_ge - sum_gt) * (need / eq_cnt)
    out_ref[0] = row_sum


def _emb_body(size_ref, denom_ref, asum_ref, anc_ref, w1_ref, b1_ref,
              w2_ref, b2_ref, out_ref):
    size_f = size_ref[0, 0]
    denom = denom_ref[0, 0]
    asum = asum_ref[...]
    anc = anc_ref[...]
    C = asum.shape[1]
    dn = functools.partial(
        jax.lax.dot_general,
        dimension_numbers=(((1,), (1,)), ((), ())),
        preferred_element_type=jnp.float32,
        precision=jax.lax.Precision.HIGHEST,
    )
    emb1 = (dn(asum, w1_ref[...]) + size_f * b1_ref[...]) / denom
    w2a = w2_ref[:, :C]
    w2b = w2_ref[:, C:]
    out_ref[...] = dn(anc, w2a) + dn(emb1, w2b) + b2_ref[...]


def kernel(clsfea, denfea, anchor_cls, bs_mean, b, W_emb, b_emb, W_emb2,
           b_emb2, interpret=False):
    Bc, C, H, Wd = clsfea.shape
    K = H * Wd
    a2 = 384.0 * 576.0 / H / Wd
    prod = bs_mean[0, 0] * bs_mean[0, 1]
    size = jnp.floor_divide(prod.astype(jnp.float32),
                            jnp.float32(a2)).astype(jnp.int32)
    size = jnp.maximum(size, 3)
    size_f = size.astype(jnp.float32).reshape(1, 1)
    denom = (size.astype(jnp.float32)
             * (jnp.asarray(b, jnp.float32) / Bc)).reshape(1, 1)

    cls3 = clsfea.reshape(Bc, C, K)
    den3 = denfea.reshape(Bc, C, K)
    anc3 = anchor_cls.reshape(Bc, 1, C)

    asum = pl.pallas_call(
        _select_body,
        grid=(Bc,),
        in_specs=[
            pl.BlockSpec((1, 1), lambda i: (0, 0)),
            pl.BlockSpec((1, C, K), lambda i: (i, 0, 0)),
            pl.BlockSpec((1, C, K), lambda i: (i, 0, 0)),
            pl.BlockSpec((1, 1, C), lambda i: (i, 0, 0)),
        ],
        out_specs=pl.BlockSpec((1, C, 1), lambda i: (i, 0, 0)),
        out_shape=jax.ShapeDtypeStruct((Bc, C, 1), jnp.float32),
        interpret=interpret,
    )(size_f, cls3, den3, anc3)

    out = pl.pallas_call(
        _emb_body,
        out_shape=jax.ShapeDtypeStruct((Bc, C), jnp.float32),
        interpret=interpret,
    )(size_f, denom, asum.reshape(Bc, C), anchor_cls.reshape(Bc, C),
      W_emb, b_emb.reshape(1, C), W_emb2, b_emb2.reshape(1, C))

    return out.reshape(Bc, C, 1, 1)
